# baseline (speedup 1.0000x reference)
import functools

import jax
import jax.numpy as jnp
from jax.experimental import pallas as pl
from jax.experimental.pallas import tpu as pltpu

_EMBED = 64
_TILE = 256


def _copy_plan():
    plan = []
    for j in range(13):
        if j % 2 == 0:
            plan.append((0, j, 0, j // 2))
            plan.append((1, j, 1, 6 + j // 2))
        else:
            plan.append((0, j, 1, (j - 1) // 2))
            plan.append((1, j, 0, 7 + (j - 1) // 2))
    return plan


def _body(v0_ref, v1_ref, ev_ref, od_ref):
    srcs = (v0_ref, v1_ref)
    dsts = (ev_ref, od_ref)
    for si, sb, di, db in _copy_plan():
        dsts[di][:, db * _EMBED:(db + 1) * _EMBED] = (
            srcs[si][:, sb * _EMBED:(sb + 1) * _EMBED])


def kernel(values0, values1):
    B, W = values0.shape
    grid = (B // _TILE,)
    spec = pl.BlockSpec((_TILE, W), lambda i: (i, 0))
    out_t = (
        jax.ShapeDtypeStruct((B, W), jnp.float32),
        jax.ShapeDtypeStruct((B, W), jnp.float32),
    )
    return pl.pallas_call(
        _body,
        grid=grid,
        in_specs=[spec, spec],
        out_specs=(spec, spec),
        out_shape=out_t,
    )(values0, values1)


# speedup vs baseline: 1.0644x; 1.0644x over previous
import functools

import jax
import jax.numpy as jnp
from jax.experimental import pallas as pl
from jax.experimental.pallas import tpu as pltpu

_EMBED = 64
_TILE = 2048


def _copy_plan():
    plan = []
    for j in range(13):
        if j % 2 == 0:
            plan.append((0, j, 0, j // 2))
            plan.append((1, j, 1, 6 + j // 2))
        else:
            plan.append((0, j, 1, (j - 1) // 2))
            plan.append((1, j, 0, 7 + (j - 1) // 2))
    return plan


def _body(v0_ref, v1_ref, ev_ref, od_ref):
    srcs = (v0_ref, v1_ref)
    dsts = (ev_ref, od_ref)
    for si, sb, di, db in _copy_plan():
        dsts[di][:, db * _EMBED:(db + 1) * _EMBED] = (
            srcs[si][:, sb * _EMBED:(sb + 1) * _EMBED])


def kernel(values0, values1):
    B, W = values0.shape
    grid = (B // _TILE,)
    spec = pl.BlockSpec((_TILE, W), lambda i: (i, 0))
    out_t = (
        jax.ShapeDtypeStruct((B, W), jnp.float32),
        jax.ShapeDtypeStruct((B, W), jnp.float32),
    )
    return pl.pallas_call(
        _body,
        grid=grid,
        in_specs=[spec, spec],
        out_specs=(spec, spec),
        out_shape=out_t,
    )(values0, values1)
